# baseline (speedup 1.0000x reference)
import jax
import jax.numpy as jnp
from jax import lax
from jax.experimental import pallas as pl
from jax.experimental.pallas import tpu as pltpu
from jax.experimental.pallas import tpu_sc as plsc

N_PROTO = 16
LATENT = 1024
OUT = 1000
N_SUB = 64
BATCH = 8192
OUTP = 1024
BIG = 1e30

TM = 128
NT = (BATCH + N_PROTO * (TM - 1) + TM - 1) // TM
P = NT * TM
NS = 16
NC = 2
NW = NS * NC
RPW = BATCH // NW
RPB = BATCH // NW
CH = 32
NCH = RPB // CH


_LANE = None


def _take16(v, idx):
    return lax.gather(
        v, idx[:, None],
        lax.GatherDimensionNumbers(offset_dims=(), collapsed_slice_dims=(0,),
                                   start_index_map=(0,)),
        (1,), mode=lax.GatherScatterMode.PROMISE_IN_BOUNDS)


def _allmin16(v, lane):
    for sh in (1, 2, 4, 8):
        v = jnp.minimum(v, _take16(v, (lane + sh) & 15))
    return v


def _allsum16(v, lane):
    for sh in (1, 2, 4, 8):
        v = v + _take16(v, (lane + sh) & 15)
    return v


def _prefsum16(v, lane):
    zero = jnp.zeros(N_PROTO, jnp.int32)
    for sh in (1, 2, 4, 8):
        shvec = jnp.full((N_PROTO,), sh, jnp.int32)
        shifted = _take16(v, jnp.maximum(lane - shvec, zero))
        v = v + jnp.where(lane >= shvec, shifted, zero)
    return v


def _count_body(x_hbm, idx_hbm, cnt_hbm, col_hbm, m1p_hbm,
                xbuf, idxbuf, cstage, fstage):
    c = lax.axis_index("c")
    s = lax.axis_index("s")
    w = s * NC + c
    lane = jnp.arange(N_PROTO, dtype=jnp.int32)

    pltpu.sync_copy(x_hbm.at[pl.ds(w * RPW * N_PROTO, RPW * N_PROTO)], xbuf)

    def grp(i, carry):
        m1acc, colmin = carry
        avec = jnp.zeros(N_PROTO, jnp.int32)
        for j in range(16):
            row = xbuf[pl.ds(i * 256 + j * 16, 16)]
            rmin = _allmin16(row, lane)
            aj = _allmin16(jnp.where(row == rmin, lane, N_PROTO), lane)
            avec = jnp.where(lane == j, aj, avec)
            m1acc = m1acc + jnp.where(lane == j, rmin, 0.0)
            colmin = jnp.minimum(colmin, row)
        idxbuf[pl.ds(i * 16, 16)] = avec
        return m1acc, colmin

    m1acc, colmin = lax.fori_loop(
        0, RPW // 16, grp,
        (jnp.zeros((N_PROTO,), jnp.float32),
         jnp.full((N_PROTO,), BIG, jnp.float32)))

    def cnt_grp(i, cvec):
        v = idxbuf[pl.ds(i * 16, 16)]
        for e in range(N_PROTO):
            ce = _allsum16(jnp.where(v == e, 1, 0), lane)
            cvec = cvec + jnp.where(lane == e, ce, 0)
        return cvec

    cnt = lax.fori_loop(0, RPW // 16, cnt_grp, jnp.zeros(N_PROTO, jnp.int32))

    pltpu.sync_copy(idxbuf, idx_hbm.at[pl.ds(w * RPW, RPW)])
    cstage[...] = cnt
    pltpu.sync_copy(cstage, cnt_hbm.at[w])
    fstage[...] = colmin
    pltpu.sync_copy(fstage, col_hbm.at[w])
    fstage[...] = m1acc
    pltpu.sync_copy(fstage, m1p_hbm.at[w])


_count_call = pl.kernel(
    _count_body,
    out_type=(jax.ShapeDtypeStruct((BATCH,), jnp.int32),
              jax.ShapeDtypeStruct((NW, N_PROTO), jnp.int32),
              jax.ShapeDtypeStruct((NW, N_PROTO), jnp.float32),
              jax.ShapeDtypeStruct((NW, N_PROTO), jnp.float32)),
    mesh=plsc.VectorSubcoreMesh(core_axis_name="c", subcore_axis_name="s"),
    scratch_types=[
        pltpu.VMEM((RPW * N_PROTO,), jnp.float32),
        pltpu.VMEM((RPW,), jnp.int32),
        pltpu.VMEM((N_PROTO,), jnp.int32),
        pltpu.VMEM((N_PROTO,), jnp.float32),
    ])


def _rank_body(idx_hbm, cnt_hbm, col_hbm, m1p_hbm,
               pos_hbm, ts_hbm, g_hbm, m1_hbm, m2_hbm,
               idxbuf, posbuf, allcnt, colbuf, m1buf, cstage, fstage):
    c = lax.axis_index("c")
    s = lax.axis_index("s")
    w = s * NC + c
    lane = jnp.arange(N_PROTO, dtype=jnp.int32)

    pltpu.sync_copy(cnt_hbm, allcnt)
    pltpu.sync_copy(idx_hbm.at[pl.ds(w * RPW, RPW)], idxbuf)

    base_vec = jnp.zeros(N_PROTO, jnp.int32)
    g_vec = jnp.zeros(N_PROTO, jnp.int32)
    zero = jnp.zeros(N_PROTO, jnp.int32)
    one = jnp.full((N_PROTO,), 1, jnp.int32)
    wvec = zero + w
    for wp in range(NW):
        row = allcnt[wp]
        g_vec = g_vec + row
        wpvec = jnp.full((N_PROTO,), wp, jnp.int32)
        mint = jnp.minimum(jnp.maximum(wvec - wpvec, zero), one)
        base_vec = base_vec + row * mint
    tmsh = jnp.full((N_PROTO,), TM.bit_length() - 1, jnp.int32)
    pg = ((g_vec + (TM - 1)) >> tmsh) << tmsh
    start = _prefsum16(pg, lane) - pg
    ts_vec = start >> tmsh
    db_vec = start + base_vec

    def rnk(i, db_vec):
        v = idxbuf[pl.ds(i * 16, 16)]
        dest = jnp.zeros(N_PROTO, jnp.int32)
        for e in range(N_PROTO):
            mi = jnp.where(v == e, 1, 0)
            pc = _prefsum16(mi, lane)
            base_e = _take16(db_vec, jnp.zeros(N_PROTO, jnp.int32) + e)
            dest = dest + mi * (base_e + pc - 1 - dest)
            ce = _take16(pc, jnp.full((N_PROTO,), 15, jnp.int32))
            db_vec = db_vec + jnp.where(lane == e, ce, 0)
        posbuf[pl.ds(i * 16, 16)] = dest
        return db_vec

    lax.fori_loop(0, RPW // 16, rnk, db_vec)
    pltpu.sync_copy(posbuf, pos_hbm.at[pl.ds(w * RPW, RPW)])

    @pl.when(jnp.logical_and(c == 0, s == 0))
    def _finalize():
        pltpu.sync_copy(col_hbm, colbuf)
        pltpu.sync_copy(m1p_hbm, m1buf)
        colT = jnp.full((N_PROTO,), BIG, jnp.float32)
        m1s = jnp.zeros((N_PROTO,), jnp.float32)
        for wp in range(NW):
            colT = jnp.minimum(colT, colbuf[wp])
            m1s = m1s + m1buf[wp]
        min1 = _allsum16(m1s, lane) * jnp.full((N_PROTO,), 1.0 / BATCH,
                                               jnp.float32)
        min2 = _allsum16(colT, lane) * jnp.full((N_PROTO,), 1.0 / N_PROTO,
                                                jnp.float32)
        cstage[...] = ts_vec
        pltpu.sync_copy(cstage, ts_hbm)
        cstage[...] = g_vec
        pltpu.sync_copy(cstage, g_hbm)
        fstage[...] = min1
        pltpu.sync_copy(fstage, m1_hbm)
        fstage[...] = min2
        pltpu.sync_copy(fstage, m2_hbm)


_rank_call = pl.kernel(
    _rank_body,
    out_type=(jax.ShapeDtypeStruct((BATCH,), jnp.int32),
              jax.ShapeDtypeStruct((N_PROTO,), jnp.int32),
              jax.ShapeDtypeStruct((N_PROTO,), jnp.int32),
              jax.ShapeDtypeStruct((N_PROTO,), jnp.float32),
              jax.ShapeDtypeStruct((N_PROTO,), jnp.float32)),
    mesh=plsc.VectorSubcoreMesh(core_axis_name="c", subcore_axis_name="s"),
    scratch_types=[
        pltpu.VMEM((RPW,), jnp.int32),
        pltpu.VMEM((RPW,), jnp.int32),
        pltpu.VMEM((NW, N_PROTO), jnp.int32),
        pltpu.VMEM((NW, N_PROTO), jnp.float32),
        pltpu.VMEM((NW, N_PROTO), jnp.float32),
        pltpu.VMEM((N_PROTO,), jnp.int32),
        pltpu.VMEM((N_PROTO,), jnp.float32),
    ])


def _disp_body(inp_hbm, pos_hbm, srt_hbm, posv, buf0, buf1, sem):
    c = lax.axis_index("c")
    s = lax.axis_index("s")
    w = s * NC + c
    bufs = (buf0, buf1)
    pltpu.sync_copy(pos_hbm.at[w], posv)
    pltpu.sync_copy(inp_hbm.at[pl.ds(w * RPB, CH)], buf0)
    for ch in range(NCH):
        scat = pltpu.async_copy(bufs[ch % 2], srt_hbm.at[posv.at[ch]], sem)
        if ch + 1 < NCH:
            pltpu.sync_copy(
                inp_hbm.at[pl.ds(w * RPB + (ch + 1) * CH, CH)],
                bufs[(ch + 1) % 2])
        scat.wait()


_disp_call = pl.kernel(
    _disp_body,
    out_type=jax.ShapeDtypeStruct((P, LATENT), jnp.float32),
    mesh=plsc.VectorSubcoreMesh(core_axis_name="c", subcore_axis_name="s"),
    scratch_types=[
        pltpu.VMEM((NCH, CH), jnp.int32),
        pltpu.VMEM((CH, LATENT), jnp.float32),
        pltpu.VMEM((CH, LATENT), jnp.float32),
        pltpu.SemaphoreType.DMA,
    ])


def _comb_body(osrt_hbm, pos_hbm, out_hbm, posv, buf0, buf1, sem0, sem1):
    c = lax.axis_index("c")
    s = lax.axis_index("s")
    w = s * NC + c
    bufs = (buf0, buf1)
    sems = (sem0, sem1)
    pltpu.sync_copy(pos_hbm.at[w], posv)
    pltpu.async_copy(osrt_hbm.at[posv.at[0]], buf0, sem0)
    for ch in range(NCH):
        pltpu.make_async_copy(
            osrt_hbm.at[posv.at[ch]], bufs[ch % 2], sems[ch % 2]).wait()
        if ch + 1 < NCH:
            pltpu.async_copy(osrt_hbm.at[posv.at[ch + 1]],
                             bufs[(ch + 1) % 2], sems[(ch + 1) % 2])
        pltpu.sync_copy(bufs[ch % 2],
                        out_hbm.at[pl.ds(w * RPB + ch * CH, CH)])


_comb_call = pl.kernel(
    _comb_body,
    out_type=jax.ShapeDtypeStruct((BATCH, OUTP), jnp.float32),
    mesh=plsc.VectorSubcoreMesh(core_axis_name="c", subcore_axis_name="s"),
    scratch_types=[
        pltpu.VMEM((NCH, CH), jnp.int32),
        pltpu.VMEM((CH, OUTP), jnp.float32),
        pltpu.VMEM((CH, OUTP), jnp.float32),
        pltpu.SemaphoreType.DMA,
        pltpu.SemaphoreType.DMA,
    ])


def _expert_of(t, ts_ref):
    e = jnp.int32(-1)
    for i in range(N_PROTO):
        e = e + jnp.where(ts_ref[i] <= t, 1, 0)
    return e


def _experts_body(ts_ref, g_ref, srt_ref, sp_ref, w_ref, b_ref,
                  osrt_ref, sub1_ref, sub2_ref, minv_ref, s2_ref):
    t = pl.program_id(0)

    @pl.when(t == 0)
    def _init():
        minv_ref[...] = jnp.full_like(minv_ref, BIG)
        s2_ref[...] = jnp.zeros_like(s2_ref)

    e = _expert_of(t, ts_ref)
    valid_n = g_ref[e] - (t - ts_ref[e]) * TM
    rowi = lax.broadcasted_iota(jnp.int32, (TM, 1), 0)
    vmask = rowi < valid_n

    @pl.when(valid_n > 0)
    def _compute():
        xf = srt_ref[...]
        sp = sp_ref[0]
        x2 = jnp.sum(xf * xf, axis=1, keepdims=True)
        sp2 = jnp.sum(sp * sp, axis=1)[None, :]
        tdot = lax.dot_general(
            xf.astype(jnp.bfloat16), sp.astype(jnp.bfloat16),
            (((1,), (1,)), ((), ())), preferred_element_type=jnp.float32)
        d = x2 - 2.0 * tdot + sp2

        rmind = jnp.min(d, axis=1, keepdims=True)
        elane = lax.broadcasted_iota(jnp.int32, (1, N_PROTO), 1)
        s2_ref[...] += jnp.where(
            elane == e, jnp.sum(jnp.where(vmask, rmind, 0.0)), 0.0)
        dm = jnp.where(vmask, d, BIG)
        newmin = jnp.min(dm, axis=0)[None, :]
        esub = lax.broadcasted_iota(jnp.int32, minv_ref.shape, 0)
        minv_ref[...] = jnp.where(
            esub == e, jnp.minimum(minv_ref[...], newmin), minv_ref[...])

        o = lax.dot_general(
            d.astype(jnp.bfloat16), w_ref[0].astype(jnp.bfloat16),
            (((1,), (1,)), ((), ())), preferred_element_type=jnp.float32)
        osrt_ref[...] = o + b_ref[0]

    @pl.when(t == pl.num_programs(0) - 1)
    def _finalize():
        elane = lax.broadcasted_iota(jnp.int32, (1, N_PROTO), 1)
        gvec = jnp.zeros((1, N_PROTO), jnp.float32)
        for i in range(N_PROTO):
            gvec = gvec + jnp.where(elane == i,
                                    g_ref[i].astype(jnp.float32), 0.0)
        nonempty = gvec > 0.0
        m1 = jnp.sum(minv_ref[...], axis=1)[None, :] / N_SUB
        sub1_ref[...] = (jnp.sum(jnp.where(nonempty, m1, 0.0))
                         / N_PROTO).reshape(1, 1)
        m2 = s2_ref[...] / jnp.maximum(gvec, 1.0)
        sub2_ref[...] = (jnp.sum(jnp.where(nonempty, m2, 0.0))
                         / N_PROTO).reshape(1, 1)


def _experts_call(ts, g, srt, sp, w, b3):
    grid_spec = pltpu.PrefetchScalarGridSpec(
        num_scalar_prefetch=2,
        grid=(NT,),
        in_specs=[
            pl.BlockSpec((TM, LATENT), lambda t, ts, g: (t, 0)),
            pl.BlockSpec((1, N_SUB, LATENT),
                         lambda t, ts, g: (_expert_of(t, ts), 0, 0)),
            pl.BlockSpec((1, OUTP, N_SUB),
                         lambda t, ts, g: (_expert_of(t, ts), 0, 0)),
            pl.BlockSpec((1, 1, OUTP),
                         lambda t, ts, g: (_expert_of(t, ts), 0, 0)),
        ],
        out_specs=(
            pl.BlockSpec((TM, OUTP), lambda t, ts, g: (t, 0)),
            pl.BlockSpec((1, 1), lambda t, ts, g: (0, 0)),
            pl.BlockSpec((1, 1), lambda t, ts, g: (0, 0)),
        ),
        scratch_shapes=[
            pltpu.VMEM((N_PROTO, N_SUB), jnp.float32),
            pltpu.VMEM((1, N_PROTO), jnp.float32),
        ],
    )
    return pl.pallas_call(
        _experts_body,
        grid_spec=grid_spec,
        out_shape=(jax.ShapeDtypeStruct((P, OUTP), jnp.float32),
                   jax.ShapeDtypeStruct((1, 1), jnp.float32),
                   jax.ShapeDtypeStruct((1, 1), jnp.float32)),
    )(ts, g, srt, sp, w, b3)


def kernel(input, prototypes, sub_prototypes, lin_w, lin_b):
    inp = input.astype(jnp.float32).reshape(input.shape[0], LATENT)
    x = (jnp.sum(inp * inp, axis=1, keepdims=True)
         - 2.0 * inp @ prototypes.T
         + jnp.sum(prototypes * prototypes, axis=1)[None, :])
    idx, lcnt, lcol, lm1 = _count_call(x.reshape(-1))
    pos, ts, g, m1v, m2v = _rank_call(idx, lcnt, lcol, lm1)
    pos3 = pos.reshape(NW, NCH, CH)
    srt = _disp_call(inp, pos3)
    wp = jnp.pad(lin_w, ((0, 0), (0, OUTP - OUT), (0, 0)))
    bp = jnp.pad(lin_b, ((0, 0), (0, OUTP - OUT))).reshape(N_PROTO, 1, OUTP)
    osrt, sub1, sub2 = _experts_call(ts, g, srt, sub_prototypes, wp, bp)
    outp = _comb_call(osrt, pos3)
    return (m1v[0], m2v[0], sub1.reshape(()), sub2.reshape(()),
            outp[:, :OUT])


# speedup vs baseline: 1.0076x; 1.0076x over previous
import jax
import jax.numpy as jnp
from jax import lax
from jax.experimental import pallas as pl
from jax.experimental.pallas import tpu as pltpu
from jax.experimental.pallas import tpu_sc as plsc

N_PROTO = 16
LATENT = 1024
OUT = 1000
N_SUB = 64
BATCH = 8192
OUTP = 1024
BIG = 1e30

TM = 128
NT = (BATCH + N_PROTO * (TM - 1) + TM - 1) // TM
P = NT * TM
NS = 16
NC = 2
NW = NS * NC
RPW = BATCH // NW
RPB = BATCH // NW
CH = 32
NCH = RPB // CH
DCH = 16
NDCH = RPB // DCH


_LANE = None


def _take16(v, idx):
    return lax.gather(
        v, idx[:, None],
        lax.GatherDimensionNumbers(offset_dims=(), collapsed_slice_dims=(0,),
                                   start_index_map=(0,)),
        (1,), mode=lax.GatherScatterMode.PROMISE_IN_BOUNDS)


def _allmin16(v, lane):
    for sh in (1, 2, 4, 8):
        v = jnp.minimum(v, _take16(v, (lane + sh) & 15))
    return v


def _allsum16(v, lane):
    for sh in (1, 2, 4, 8):
        v = v + _take16(v, (lane + sh) & 15)
    return v


def _prefsum16(v, lane):
    zero = jnp.zeros(N_PROTO, jnp.int32)
    for sh in (1, 2, 4, 8):
        shvec = jnp.full((N_PROTO,), sh, jnp.int32)
        shifted = _take16(v, jnp.maximum(lane - shvec, zero))
        v = v + jnp.where(lane >= shvec, shifted, zero)
    return v


def _count_body(x_hbm, idx_hbm, cnt_hbm, col_hbm, m1p_hbm,
                xbuf, idxbuf, cstage, fstage):
    c = lax.axis_index("c")
    s = lax.axis_index("s")
    w = s * NC + c
    lane = jnp.arange(N_PROTO, dtype=jnp.int32)

    pltpu.sync_copy(x_hbm.at[pl.ds(w * RPW * N_PROTO, RPW * N_PROTO)], xbuf)

    def grp(i, carry):
        m1acc, colmin = carry
        avec = jnp.zeros(N_PROTO, jnp.int32)
        for j in range(16):
            row = xbuf[pl.ds(i * 256 + j * 16, 16)]
            rmin = _allmin16(row, lane)
            aj = _allmin16(jnp.where(row == rmin, lane, N_PROTO), lane)
            avec = jnp.where(lane == j, aj, avec)
            m1acc = m1acc + jnp.where(lane == j, rmin, 0.0)
            colmin = jnp.minimum(colmin, row)
        idxbuf[pl.ds(i * 16, 16)] = avec
        return m1acc, colmin

    m1acc, colmin = lax.fori_loop(
        0, RPW // 16, grp,
        (jnp.zeros((N_PROTO,), jnp.float32),
         jnp.full((N_PROTO,), BIG, jnp.float32)))

    def cnt_grp(i, cvec):
        v = idxbuf[pl.ds(i * 16, 16)]
        for e in range(N_PROTO):
            ce = _allsum16(jnp.where(v == e, 1, 0), lane)
            cvec = cvec + jnp.where(lane == e, ce, 0)
        return cvec

    cnt = lax.fori_loop(0, RPW // 16, cnt_grp, jnp.zeros(N_PROTO, jnp.int32))

    pltpu.sync_copy(idxbuf, idx_hbm.at[pl.ds(w * RPW, RPW)])
    cstage[...] = cnt
    pltpu.sync_copy(cstage, cnt_hbm.at[w])
    fstage[...] = colmin
    pltpu.sync_copy(fstage, col_hbm.at[w])
    fstage[...] = m1acc
    pltpu.sync_copy(fstage, m1p_hbm.at[w])


_count_call = pl.kernel(
    _count_body,
    out_type=(jax.ShapeDtypeStruct((BATCH,), jnp.int32),
              jax.ShapeDtypeStruct((NW, N_PROTO), jnp.int32),
              jax.ShapeDtypeStruct((NW, N_PROTO), jnp.float32),
              jax.ShapeDtypeStruct((NW, N_PROTO), jnp.float32)),
    mesh=plsc.VectorSubcoreMesh(core_axis_name="c", subcore_axis_name="s"),
    scratch_types=[
        pltpu.VMEM((RPW * N_PROTO,), jnp.float32),
        pltpu.VMEM((RPW,), jnp.int32),
        pltpu.VMEM((N_PROTO,), jnp.int32),
        pltpu.VMEM((N_PROTO,), jnp.float32),
    ])


def _rank_body(idx_hbm, cnt_hbm, col_hbm, m1p_hbm, inp_hbm,
               pos_hbm, ts_hbm, g_hbm, m1_hbm, m2_hbm, srt_hbm,
               idxbuf, posv, allcnt, colbuf, m1buf, cstage, fstage,
               buf0, buf1, sem):
    c = lax.axis_index("c")
    s = lax.axis_index("s")
    w = s * NC + c
    lane = jnp.arange(N_PROTO, dtype=jnp.int32)

    pltpu.sync_copy(cnt_hbm, allcnt)
    pltpu.sync_copy(idx_hbm.at[pl.ds(w * RPW, RPW)], idxbuf)

    base_vec = jnp.zeros(N_PROTO, jnp.int32)
    g_vec = jnp.zeros(N_PROTO, jnp.int32)
    zero = jnp.zeros(N_PROTO, jnp.int32)
    one = jnp.full((N_PROTO,), 1, jnp.int32)
    wvec = zero + w
    for wp in range(NW):
        row = allcnt[wp]
        g_vec = g_vec + row
        wpvec = jnp.full((N_PROTO,), wp, jnp.int32)
        mint = jnp.minimum(jnp.maximum(wvec - wpvec, zero), one)
        base_vec = base_vec + row * mint
    tmsh = jnp.full((N_PROTO,), TM.bit_length() - 1, jnp.int32)
    pg = ((g_vec + (TM - 1)) >> tmsh) << tmsh
    start = _prefsum16(pg, lane) - pg
    ts_vec = start >> tmsh
    db_vec = start + base_vec

    def rnk(i, db_vec):
        v = idxbuf[pl.ds(i * 16, 16)]
        dest = jnp.zeros(N_PROTO, jnp.int32)
        for e in range(N_PROTO):
            mi = jnp.where(v == e, 1, 0)
            pc = _prefsum16(mi, lane)
            base_e = _take16(db_vec, jnp.zeros(N_PROTO, jnp.int32) + e)
            dest = dest + mi * (base_e + pc - 1 - dest)
            ce = _take16(pc, jnp.full((N_PROTO,), 15, jnp.int32))
            db_vec = db_vec + jnp.where(lane == e, ce, 0)
        posv[i] = dest
        return db_vec

    lax.fori_loop(0, RPW // 16, rnk, db_vec)
    pltpu.sync_copy(posv, pos_hbm.at[w])

    bufs = (buf0, buf1)
    pltpu.sync_copy(inp_hbm.at[pl.ds(w * RPB, DCH)], buf0)
    for ch in range(NDCH):
        scat = pltpu.async_copy(bufs[ch % 2], srt_hbm.at[posv.at[ch]], sem)
        if ch + 1 < NDCH:
            pltpu.sync_copy(
                inp_hbm.at[pl.ds(w * RPB + (ch + 1) * DCH, DCH)],
                bufs[(ch + 1) % 2])
        scat.wait()

    @pl.when(jnp.logical_and(c == 0, s == 0))
    def _finalize():
        pltpu.sync_copy(col_hbm, colbuf)
        pltpu.sync_copy(m1p_hbm, m1buf)
        colT = jnp.full((N_PROTO,), BIG, jnp.float32)
        m1s = jnp.zeros((N_PROTO,), jnp.float32)
        for wp in range(NW):
            colT = jnp.minimum(colT, colbuf[wp])
            m1s = m1s + m1buf[wp]
        min1 = _allsum16(m1s, lane) * jnp.full((N_PROTO,), 1.0 / BATCH,
                                               jnp.float32)
        min2 = _allsum16(colT, lane) * jnp.full((N_PROTO,), 1.0 / N_PROTO,
                                                jnp.float32)
        cstage[...] = ts_vec
        pltpu.sync_copy(cstage, ts_hbm)
        cstage[...] = g_vec
        pltpu.sync_copy(cstage, g_hbm)
        fstage[...] = min1
        pltpu.sync_copy(fstage, m1_hbm)
        fstage[...] = min2
        pltpu.sync_copy(fstage, m2_hbm)


_rank_call = pl.kernel(
    _rank_body,
    out_type=(jax.ShapeDtypeStruct((NW, NDCH, DCH), jnp.int32),
              jax.ShapeDtypeStruct((N_PROTO,), jnp.int32),
              jax.ShapeDtypeStruct((N_PROTO,), jnp.int32),
              jax.ShapeDtypeStruct((N_PROTO,), jnp.float32),
              jax.ShapeDtypeStruct((N_PROTO,), jnp.float32),
              jax.ShapeDtypeStruct((P, LATENT), jnp.float32)),
    mesh=plsc.VectorSubcoreMesh(core_axis_name="c", subcore_axis_name="s"),
    scratch_types=[
        pltpu.VMEM((RPW,), jnp.int32),
        pltpu.VMEM((NDCH, DCH), jnp.int32),
        pltpu.VMEM((NW, N_PROTO), jnp.int32),
        pltpu.VMEM((NW, N_PROTO), jnp.float32),
        pltpu.VMEM((NW, N_PROTO), jnp.float32),
        pltpu.VMEM((N_PROTO,), jnp.int32),
        pltpu.VMEM((N_PROTO,), jnp.float32),
        pltpu.VMEM((DCH, LATENT), jnp.float32),
        pltpu.VMEM((DCH, LATENT), jnp.float32),
        pltpu.SemaphoreType.DMA,
    ])


def _disp_body(inp_hbm, pos_hbm, srt_hbm, posv, buf0, buf1, sem):
    c = lax.axis_index("c")
    s = lax.axis_index("s")
    w = s * NC + c
    bufs = (buf0, buf1)
    pltpu.sync_copy(pos_hbm.at[w], posv)
    pltpu.sync_copy(inp_hbm.at[pl.ds(w * RPB, CH)], buf0)
    for ch in range(NCH):
        scat = pltpu.async_copy(bufs[ch % 2], srt_hbm.at[posv.at[ch]], sem)
        if ch + 1 < NCH:
            pltpu.sync_copy(
                inp_hbm.at[pl.ds(w * RPB + (ch + 1) * CH, CH)],
                bufs[(ch + 1) % 2])
        scat.wait()


_disp_call = pl.kernel(
    _disp_body,
    out_type=jax.ShapeDtypeStruct((P, LATENT), jnp.float32),
    mesh=plsc.VectorSubcoreMesh(core_axis_name="c", subcore_axis_name="s"),
    scratch_types=[
        pltpu.VMEM((NCH, CH), jnp.int32),
        pltpu.VMEM((CH, LATENT), jnp.float32),
        pltpu.VMEM((CH, LATENT), jnp.float32),
        pltpu.SemaphoreType.DMA,
    ])


def _comb_body(osrt_hbm, pos_hbm, out_hbm, posv, buf0, buf1, sem0, sem1):
    c = lax.axis_index("c")
    s = lax.axis_index("s")
    w = s * NC + c
    bufs = (buf0, buf1)
    sems = (sem0, sem1)
    pltpu.sync_copy(pos_hbm.at[w], posv)
    pltpu.async_copy(osrt_hbm.at[posv.at[0]], buf0, sem0)
    for ch in range(NCH):
        pltpu.make_async_copy(
            osrt_hbm.at[posv.at[ch]], bufs[ch % 2], sems[ch % 2]).wait()
        if ch + 1 < NCH:
            pltpu.async_copy(osrt_hbm.at[posv.at[ch + 1]],
                             bufs[(ch + 1) % 2], sems[(ch + 1) % 2])
        pltpu.sync_copy(bufs[ch % 2],
                        out_hbm.at[pl.ds(w * RPB + ch * CH, CH)])


_comb_call = pl.kernel(
    _comb_body,
    out_type=jax.ShapeDtypeStruct((BATCH, OUTP), jnp.float32),
    mesh=plsc.VectorSubcoreMesh(core_axis_name="c", subcore_axis_name="s"),
    scratch_types=[
        pltpu.VMEM((NCH, CH), jnp.int32),
        pltpu.VMEM((CH, OUTP), jnp.float32),
        pltpu.VMEM((CH, OUTP), jnp.float32),
        pltpu.SemaphoreType.DMA,
        pltpu.SemaphoreType.DMA,
    ])


def _expert_of(t, ts_ref):
    e = jnp.int32(-1)
    for i in range(N_PROTO):
        e = e + jnp.where(ts_ref[i] <= t, 1, 0)
    return e


def _experts_body(ts_ref, g_ref, srt_ref, sp_ref, w_ref, b_ref,
                  osrt_ref, sub1_ref, sub2_ref, minv_ref, s2_ref):
    t = pl.program_id(0)

    @pl.when(t == 0)
    def _init():
        minv_ref[...] = jnp.full_like(minv_ref, BIG)
        s2_ref[...] = jnp.zeros_like(s2_ref)

    e = _expert_of(t, ts_ref)
    valid_n = g_ref[e] - (t - ts_ref[e]) * TM
    rowi = lax.broadcasted_iota(jnp.int32, (TM, 1), 0)
    vmask = rowi < valid_n

    @pl.when(valid_n > 0)
    def _compute():
        xf = srt_ref[...]
        sp = sp_ref[0]
        x2 = jnp.sum(xf * xf, axis=1, keepdims=True)
        sp2 = jnp.sum(sp * sp, axis=1)[None, :]
        tdot = lax.dot_general(
            xf.astype(jnp.bfloat16), sp.astype(jnp.bfloat16),
            (((1,), (1,)), ((), ())), preferred_element_type=jnp.float32)
        d = x2 - 2.0 * tdot + sp2

        rmind = jnp.min(d, axis=1, keepdims=True)
        elane = lax.broadcasted_iota(jnp.int32, (1, N_PROTO), 1)
        s2_ref[...] += jnp.where(
            elane == e, jnp.sum(jnp.where(vmask, rmind, 0.0)), 0.0)
        dm = jnp.where(vmask, d, BIG)
        newmin = jnp.min(dm, axis=0)[None, :]
        esub = lax.broadcasted_iota(jnp.int32, minv_ref.shape, 0)
        minv_ref[...] = jnp.where(
            esub == e, jnp.minimum(minv_ref[...], newmin), minv_ref[...])

        o = lax.dot_general(
            d.astype(jnp.bfloat16), w_ref[0].astype(jnp.bfloat16),
            (((1,), (1,)), ((), ())), preferred_element_type=jnp.float32)
        osrt_ref[...] = o + b_ref[0]

    @pl.when(t == pl.num_programs(0) - 1)
    def _finalize():
        elane = lax.broadcasted_iota(jnp.int32, (1, N_PROTO), 1)
        gvec = jnp.zeros((1, N_PROTO), jnp.float32)
        for i in range(N_PROTO):
            gvec = gvec + jnp.where(elane == i,
                                    g_ref[i].astype(jnp.float32), 0.0)
        nonempty = gvec > 0.0
        m1 = jnp.sum(minv_ref[...], axis=1)[None, :] / N_SUB
        sub1_ref[...] = (jnp.sum(jnp.where(nonempty, m1, 0.0))
                         / N_PROTO).reshape(1, 1)
        m2 = s2_ref[...] / jnp.maximum(gvec, 1.0)
        sub2_ref[...] = (jnp.sum(jnp.where(nonempty, m2, 0.0))
                         / N_PROTO).reshape(1, 1)


def _experts_call(ts, g, srt, sp, w, b3):
    grid_spec = pltpu.PrefetchScalarGridSpec(
        num_scalar_prefetch=2,
        grid=(NT,),
        in_specs=[
            pl.BlockSpec((TM, LATENT), lambda t, ts, g: (t, 0)),
            pl.BlockSpec((1, N_SUB, LATENT),
                         lambda t, ts, g: (_expert_of(t, ts), 0, 0)),
            pl.BlockSpec((1, OUTP, N_SUB),
                         lambda t, ts, g: (_expert_of(t, ts), 0, 0)),
            pl.BlockSpec((1, 1, OUTP),
                         lambda t, ts, g: (_expert_of(t, ts), 0, 0)),
        ],
        out_specs=(
            pl.BlockSpec((TM, OUTP), lambda t, ts, g: (t, 0)),
            pl.BlockSpec((1, 1), lambda t, ts, g: (0, 0)),
            pl.BlockSpec((1, 1), lambda t, ts, g: (0, 0)),
        ),
        scratch_shapes=[
            pltpu.VMEM((N_PROTO, N_SUB), jnp.float32),
            pltpu.VMEM((1, N_PROTO), jnp.float32),
        ],
    )
    return pl.pallas_call(
        _experts_body,
        grid_spec=grid_spec,
        out_shape=(jax.ShapeDtypeStruct((P, OUTP), jnp.float32),
                   jax.ShapeDtypeStruct((1, 1), jnp.float32),
                   jax.ShapeDtypeStruct((1, 1), jnp.float32)),
    )(ts, g, srt, sp, w, b3)


def kernel(input, prototypes, sub_prototypes, lin_w, lin_b):
    inp = input.astype(jnp.float32).reshape(input.shape[0], LATENT)
    x = (jnp.sum(inp * inp, axis=1, keepdims=True)
         - 2.0 * inp @ prototypes.T
         + jnp.sum(prototypes * prototypes, axis=1)[None, :])
    idx, lcnt, lcol, lm1 = _count_call(x.reshape(-1))
    pos, ts, g, m1v, m2v, srt = _rank_call(idx, lcnt, lcol, lm1, inp)
    pos3 = pos.reshape(NW, NCH, CH)
    wp = jnp.pad(lin_w, ((0, 0), (0, OUTP - OUT), (0, 0)))
    bp = jnp.pad(lin_b, ((0, 0), (0, OUTP - OUT))).reshape(N_PROTO, 1, OUTP)
    osrt, sub1, sub2 = _experts_call(ts, g, srt, sub_prototypes, wp, bp)
    outp = _comb_call(osrt, pos3)
    return (m1v[0], m2v[0], sub1.reshape(()), sub2.reshape(()),
            outp[:, :OUT])
